# baseline (device time: 380923 ns/iter reference)
import jax
import jax.numpy as jnp
from jax import lax
from jax.experimental import pallas as pl
from jax.experimental.pallas import tpu as pltpu

C = 512
MESH = pl.DeviceIdType.MESH


def kernel(x):
    m_per, n = x.shape
    Q = m_per // 4
    K = Q // C
    H = C // 2
    assert Q % C == 0 and K >= 3

    def body(x_hbm, out_hbm,
             local_f32, zsend, zrecv, rbuf, xrecv, yrecv, dxrecv, dyrecv,
             in_sems, out_sems,
             z_ssem, z_rsem, xr_ssem, xr_rsem, yr_ssem, yr_rsem,
             xf_ssem, xf_rsem, yf_ssem, yf_rsem,
             z_cr, xr_cr, yr_cr, xf_cr, yf_cr):
        mx = lax.axis_index("x")
        my = lax.axis_index("y")
        mz = lax.axis_index("z")
        zp = (mx, my, 1 - mz)
        xn = (1 - mx, my, mz)
        yn = (mx, 1 - my, mz)
        q_me = 2 * mx + my
        q_xn = 2 * (1 - mx) + my
        q_yn = 2 * mx + (1 - my)
        q_dg = 2 * (1 - mx) + (1 - my)

        def sig(sem, nbr):
            pl.semaphore_signal(sem, inc=1, device_id=nbr,
                                device_id_type=MESH)

        barrier = pltpu.get_barrier_semaphore()
        for nbr in (zp, xn, yn):
            sig(barrier, nbr)
        pl.semaphore_wait(barrier, 3)

        zr, xr, yr, xf, yf = {}, {}, {}, {}, {}
        st_r, st_x, st_y, st_dx, st_dy = {}, {}, {}, {}, {}

        for it in range(K + 3):
            if it < K:
                k, s = it, it % 2
                load = pltpu.make_async_copy(
                    x_hbm.at[pl.ds(q_me * Q + k * C, C), :],
                    local_f32.at[s], in_sems.at[s])
                load.start()
                load.wait()
                zsend[s] = local_f32[s].astype(jnp.bfloat16)
                if k >= 2:
                    pl.semaphore_wait(z_cr, 1)
                zr[k] = pltpu.make_async_remote_copy(
                    zsend.at[s], zrecv.at[s], z_ssem.at[s], z_rsem.at[s],
                    device_id=zp, device_id_type=MESH)
                zr[k].start()

            if 0 <= it - 1 < K:
                k, s = it - 1, (it - 1) % 2
                zr[k].wait()
                rbuf[s] = (local_f32[s] + zrecv[s].astype(jnp.float32)
                           ).astype(jnp.bfloat16)
                if k + 2 < K:
                    sig(z_cr, zp)
                st_r[k] = pltpu.make_async_copy(
                    rbuf.at[s], out_hbm.at[pl.ds(q_me * Q + k * C, C), :],
                    out_sems.at[0, s])
                st_r[k].start()
                if k >= 2:
                    pl.semaphore_wait(xr_cr, 1)
                    pl.semaphore_wait(yr_cr, 1)
                xr[k] = pltpu.make_async_remote_copy(
                    rbuf.at[s], xrecv.at[s], xr_ssem.at[s], xr_rsem.at[s],
                    device_id=xn, device_id_type=MESH)
                yr[k] = pltpu.make_async_remote_copy(
                    rbuf.at[s], yrecv.at[s], yr_ssem.at[s], yr_rsem.at[s],
                    device_id=yn, device_id_type=MESH)
                xr[k].start()
                yr[k].start()

            if 0 <= it - 2 < K:
                k, s = it - 2, (it - 2) % 2
                xr[k].wait()
                yr[k].wait()
                st_x[k] = pltpu.make_async_copy(
                    xrecv.at[s], out_hbm.at[pl.ds(q_xn * Q + k * C, C), :],
                    out_sems.at[1, s])
                st_y[k] = pltpu.make_async_copy(
                    yrecv.at[s], out_hbm.at[pl.ds(q_yn * Q + k * C, C), :],
                    out_sems.at[2, s])
                st_x[k].start()
                st_y[k].start()
                if k >= 2:
                    pl.semaphore_wait(xf_cr, 1)
                    pl.semaphore_wait(yf_cr, 1)
                xf[k] = pltpu.make_async_remote_copy(
                    yrecv.at[s, pl.ds(0, H), :], dxrecv.at[s],
                    xf_ssem.at[s], xf_rsem.at[s],
                    device_id=xn, device_id_type=MESH)
                yf[k] = pltpu.make_async_remote_copy(
                    xrecv.at[s, pl.ds(H, H), :], dyrecv.at[s],
                    yf_ssem.at[s], yf_rsem.at[s],
                    device_id=yn, device_id_type=MESH)
                xf[k].start()
                yf[k].start()
                xf[k].wait_send()
                yf[k].wait_send()
                st_x[k].wait()
                st_y[k].wait()
                st_r[k].wait()
                if k + 2 < K:
                    sig(xr_cr, xn)
                    sig(yr_cr, yn)

            if 0 <= it - 3 < K:
                k, s = it - 3, (it - 3) % 2
                xf[k].wait_recv()
                yf[k].wait_recv()
                st_dx[k] = pltpu.make_async_copy(
                    dxrecv.at[s], out_hbm.at[pl.ds(q_dg * Q + k * C, H), :],
                    out_sems.at[3, s])
                st_dy[k] = pltpu.make_async_copy(
                    dyrecv.at[s],
                    out_hbm.at[pl.ds(q_dg * Q + k * C + H, H), :],
                    out_sems.at[4, s])
                st_dx[k].start()
                st_dy[k].start()
                st_dx[k].wait()
                st_dy[k].wait()
                if k + 2 < K:
                    sig(xf_cr, xn)
                    sig(yf_cr, yn)

    return pl.pallas_call(
        body,
        out_shape=jax.ShapeDtypeStruct((m_per, n), jnp.bfloat16),
        in_specs=[pl.BlockSpec(memory_space=pl.ANY)],
        out_specs=pl.BlockSpec(memory_space=pl.ANY),
        scratch_shapes=[
            pltpu.VMEM((2, C, n), jnp.float32),
            pltpu.VMEM((2, C, n), jnp.bfloat16),
            pltpu.VMEM((2, C, n), jnp.bfloat16),
            pltpu.VMEM((2, C, n), jnp.bfloat16),
            pltpu.VMEM((2, C, n), jnp.bfloat16),
            pltpu.VMEM((2, C, n), jnp.bfloat16),
            pltpu.VMEM((2, C // 2, n), jnp.bfloat16),
            pltpu.VMEM((2, C // 2, n), jnp.bfloat16),
            pltpu.SemaphoreType.DMA((2,)),
            pltpu.SemaphoreType.DMA((5, 2)),
            pltpu.SemaphoreType.DMA((2,)),
            pltpu.SemaphoreType.DMA((2,)),
            pltpu.SemaphoreType.DMA((2,)),
            pltpu.SemaphoreType.DMA((2,)),
            pltpu.SemaphoreType.DMA((2,)),
            pltpu.SemaphoreType.DMA((2,)),
            pltpu.SemaphoreType.DMA((2,)),
            pltpu.SemaphoreType.DMA((2,)),
            pltpu.SemaphoreType.DMA((2,)),
            pltpu.SemaphoreType.DMA((2,)),
            pltpu.SemaphoreType.REGULAR,
            pltpu.SemaphoreType.REGULAR,
            pltpu.SemaphoreType.REGULAR,
            pltpu.SemaphoreType.REGULAR,
            pltpu.SemaphoreType.REGULAR,
        ],
        compiler_params=pltpu.CompilerParams(
            collective_id=0, vmem_limit_bytes=100 * 1024 * 1024),
    )(x)


# device time: 334460 ns/iter; 1.1389x vs baseline; 1.1389x over previous
import jax
import jax.numpy as jnp
from jax import lax
from jax.experimental import pallas as pl
from jax.experimental.pallas import tpu as pltpu

C = 512
MESH = pl.DeviceIdType.MESH


def kernel(x):
    m_per, n = x.shape
    Q = m_per // 4
    K = Q // C
    H = C // 2
    assert Q % C == 0 and K >= 3

    def body(x_hbm, out_hbm,
             local_f32, zsend, zrecv, rbuf, xrecv, yrecv, dxrecv, dyrecv,
             in_sems, out_sems,
             z_ssem, z_rsem, xr_ssem, xr_rsem, yr_ssem, yr_rsem,
             xf_ssem, xf_rsem, yf_ssem, yf_rsem,
             z_cr, xr_cr, yr_cr, xf_cr, yf_cr):
        mx = lax.axis_index("x")
        my = lax.axis_index("y")
        mz = lax.axis_index("z")
        zp = (mx, my, 1 - mz)
        xn = (1 - mx, my, mz)
        yn = (mx, 1 - my, mz)
        q_me = 2 * mx + my
        q_xn = 2 * (1 - mx) + my
        q_yn = 2 * mx + (1 - my)
        q_dg = 2 * (1 - mx) + (1 - my)

        def sig(sem, nbr):
            pl.semaphore_signal(sem, inc=1, device_id=nbr,
                                device_id_type=MESH)

        barrier = pltpu.get_barrier_semaphore()
        for nbr in (zp, xn, yn):
            sig(barrier, nbr)
        pl.semaphore_wait(barrier, 3)

        zr, xr, yr, xf, yf = {}, {}, {}, {}, {}
        st_r, st_x, st_y, st_dx, st_dy = {}, {}, {}, {}, {}

        for it in range(K + 3):
            if it < K:
                k, s = it, it % 2
                load = pltpu.make_async_copy(
                    x_hbm.at[pl.ds(q_me * Q + k * C, C), :],
                    local_f32.at[s], in_sems.at[s])
                load.start()
                load.wait()
                zsend[s] = local_f32[s].astype(jnp.bfloat16)
                if k >= 2:
                    pl.semaphore_wait(z_cr, 1)
                zr[k] = pltpu.make_async_remote_copy(
                    zsend.at[s], zrecv.at[s], z_ssem.at[s], z_rsem.at[s],
                    device_id=zp, device_id_type=MESH)
                zr[k].start()

            if 0 <= it - 1 < K:
                k, s = it - 1, (it - 1) % 2
                zr[k].wait()
                rbuf[s] = (local_f32[s] + zrecv[s].astype(jnp.float32)
                           ).astype(jnp.bfloat16)
                if k + 2 < K:
                    sig(z_cr, zp)
                st_r[k] = pltpu.make_async_copy(
                    rbuf.at[s], out_hbm.at[pl.ds(q_me * Q + k * C, C), :],
                    out_sems.at[0, s])
                st_r[k].start()
                if k >= 3:
                    pl.semaphore_wait(xr_cr, 1)
                    pl.semaphore_wait(yr_cr, 1)
                s3 = k % 3
                xr[k] = pltpu.make_async_remote_copy(
                    rbuf.at[s], xrecv.at[s3], xr_ssem.at[s3], xr_rsem.at[s3],
                    device_id=xn, device_id_type=MESH)
                yr[k] = pltpu.make_async_remote_copy(
                    rbuf.at[s], yrecv.at[s3], yr_ssem.at[s3], yr_rsem.at[s3],
                    device_id=yn, device_id_type=MESH)
                xr[k].start()
                yr[k].start()

            if 0 <= it - 2 < K:
                k, s = it - 2, (it - 2) % 2
                s3 = k % 3
                xr[k].wait()
                yr[k].wait()
                st_x[k] = pltpu.make_async_copy(
                    xrecv.at[s3], out_hbm.at[pl.ds(q_xn * Q + k * C, C), :],
                    out_sems.at[1, s])
                st_y[k] = pltpu.make_async_copy(
                    yrecv.at[s3], out_hbm.at[pl.ds(q_yn * Q + k * C, C), :],
                    out_sems.at[2, s])
                st_x[k].start()
                st_y[k].start()
                if k >= 2:
                    pl.semaphore_wait(xf_cr, 1)
                    pl.semaphore_wait(yf_cr, 1)
                xf[k] = pltpu.make_async_remote_copy(
                    yrecv.at[s3, pl.ds(0, H), :], dxrecv.at[s],
                    xf_ssem.at[s], xf_rsem.at[s],
                    device_id=xn, device_id_type=MESH)
                yf[k] = pltpu.make_async_remote_copy(
                    xrecv.at[s3, pl.ds(H, H), :], dyrecv.at[s],
                    yf_ssem.at[s], yf_rsem.at[s],
                    device_id=yn, device_id_type=MESH)
                xf[k].start()
                yf[k].start()
                st_r[k].wait()

            if 0 <= it - 3 < K:
                k, s = it - 3, (it - 3) % 2
                xf[k].wait_recv()
                yf[k].wait_recv()
                st_dx[k] = pltpu.make_async_copy(
                    dxrecv.at[s], out_hbm.at[pl.ds(q_dg * Q + k * C, H), :],
                    out_sems.at[3, s])
                st_dy[k] = pltpu.make_async_copy(
                    dyrecv.at[s],
                    out_hbm.at[pl.ds(q_dg * Q + k * C + H, H), :],
                    out_sems.at[4, s])
                st_dx[k].start()
                st_dy[k].start()
                xf[k].wait_send()
                yf[k].wait_send()
                st_x[k].wait()
                st_y[k].wait()
                st_dx[k].wait()
                st_dy[k].wait()
                if k + 3 < K:
                    sig(xr_cr, xn)
                    sig(yr_cr, yn)
                if k + 2 < K:
                    sig(xf_cr, xn)
                    sig(yf_cr, yn)

    return pl.pallas_call(
        body,
        out_shape=jax.ShapeDtypeStruct((m_per, n), jnp.bfloat16),
        in_specs=[pl.BlockSpec(memory_space=pl.ANY)],
        out_specs=pl.BlockSpec(memory_space=pl.ANY),
        scratch_shapes=[
            pltpu.VMEM((2, C, n), jnp.float32),
            pltpu.VMEM((2, C, n), jnp.bfloat16),
            pltpu.VMEM((2, C, n), jnp.bfloat16),
            pltpu.VMEM((2, C, n), jnp.bfloat16),
            pltpu.VMEM((3, C, n), jnp.bfloat16),
            pltpu.VMEM((3, C, n), jnp.bfloat16),
            pltpu.VMEM((2, C // 2, n), jnp.bfloat16),
            pltpu.VMEM((2, C // 2, n), jnp.bfloat16),
            pltpu.SemaphoreType.DMA((2,)),
            pltpu.SemaphoreType.DMA((5, 2)),
            pltpu.SemaphoreType.DMA((2,)),
            pltpu.SemaphoreType.DMA((2,)),
            pltpu.SemaphoreType.DMA((3,)),
            pltpu.SemaphoreType.DMA((3,)),
            pltpu.SemaphoreType.DMA((3,)),
            pltpu.SemaphoreType.DMA((3,)),
            pltpu.SemaphoreType.DMA((2,)),
            pltpu.SemaphoreType.DMA((2,)),
            pltpu.SemaphoreType.DMA((2,)),
            pltpu.SemaphoreType.DMA((2,)),
            pltpu.SemaphoreType.REGULAR,
            pltpu.SemaphoreType.REGULAR,
            pltpu.SemaphoreType.REGULAR,
            pltpu.SemaphoreType.REGULAR,
            pltpu.SemaphoreType.REGULAR,
        ],
        compiler_params=pltpu.CompilerParams(
            collective_id=0, vmem_limit_bytes=100 * 1024 * 1024),
    )(x)


# device time: 328641 ns/iter; 1.1591x vs baseline; 1.0177x over previous
import jax
import jax.numpy as jnp
from jax import lax
from jax.experimental import pallas as pl
from jax.experimental.pallas import tpu as pltpu

C = 256
MESH = pl.DeviceIdType.MESH


def kernel(x):
    m_per, n = x.shape
    Q = m_per // 4
    K = Q // C
    H = C // 2
    assert Q % C == 0 and K >= 3

    def body(x_hbm, out_hbm,
             local_f32, zsend, zrecv, rbuf, xrecv, yrecv, dxrecv, dyrecv,
             in_sems, out_sems,
             z_ssem, z_rsem, xr_ssem, xr_rsem, yr_ssem, yr_rsem,
             xf_ssem, xf_rsem, yf_ssem, yf_rsem,
             z_cr, xr_cr, yr_cr, xf_cr, yf_cr):
        mx = lax.axis_index("x")
        my = lax.axis_index("y")
        mz = lax.axis_index("z")
        zp = (mx, my, 1 - mz)
        xn = (1 - mx, my, mz)
        yn = (mx, 1 - my, mz)
        q_me = 2 * mx + my
        q_xn = 2 * (1 - mx) + my
        q_yn = 2 * mx + (1 - my)
        q_dg = 2 * (1 - mx) + (1 - my)

        def sig(sem, nbr):
            pl.semaphore_signal(sem, inc=1, device_id=nbr,
                                device_id_type=MESH)

        barrier = pltpu.get_barrier_semaphore()
        for nbr in (zp, xn, yn):
            sig(barrier, nbr)
        pl.semaphore_wait(barrier, 3)

        zr, xr, yr, xf, yf = {}, {}, {}, {}, {}
        st_r, st_x, st_y, st_dx, st_dy = {}, {}, {}, {}, {}

        for it in range(K + 3):
            if it < K:
                k, s = it, it % 2
                load = pltpu.make_async_copy(
                    x_hbm.at[pl.ds(q_me * Q + k * C, C), :],
                    local_f32.at[s], in_sems.at[s])
                load.start()
                load.wait()
                zsend[s] = local_f32[s].astype(jnp.bfloat16)
                if k >= 2:
                    pl.semaphore_wait(z_cr, 1)
                zr[k] = pltpu.make_async_remote_copy(
                    zsend.at[s], zrecv.at[s], z_ssem.at[s], z_rsem.at[s],
                    device_id=zp, device_id_type=MESH)
                zr[k].start()

            if 0 <= it - 1 < K:
                k, s = it - 1, (it - 1) % 2
                zr[k].wait()
                rbuf[s] = (local_f32[s] + zrecv[s].astype(jnp.float32)
                           ).astype(jnp.bfloat16)
                if k + 2 < K:
                    sig(z_cr, zp)
                st_r[k] = pltpu.make_async_copy(
                    rbuf.at[s], out_hbm.at[pl.ds(q_me * Q + k * C, C), :],
                    out_sems.at[0, s])
                st_r[k].start()
                if k >= 3:
                    pl.semaphore_wait(xr_cr, 1)
                    pl.semaphore_wait(yr_cr, 1)
                s3 = k % 3
                xr[k] = pltpu.make_async_remote_copy(
                    rbuf.at[s], xrecv.at[s3], xr_ssem.at[s3], xr_rsem.at[s3],
                    device_id=xn, device_id_type=MESH)
                yr[k] = pltpu.make_async_remote_copy(
                    rbuf.at[s], yrecv.at[s3], yr_ssem.at[s3], yr_rsem.at[s3],
                    device_id=yn, device_id_type=MESH)
                xr[k].start()
                yr[k].start()

            if 0 <= it - 2 < K:
                k, s = it - 2, (it - 2) % 2
                s3 = k % 3
                xr[k].wait()
                yr[k].wait()
                st_x[k] = pltpu.make_async_copy(
                    xrecv.at[s3], out_hbm.at[pl.ds(q_xn * Q + k * C, C), :],
                    out_sems.at[1, s])
                st_y[k] = pltpu.make_async_copy(
                    yrecv.at[s3], out_hbm.at[pl.ds(q_yn * Q + k * C, C), :],
                    out_sems.at[2, s])
                st_x[k].start()
                st_y[k].start()
                if k >= 2:
                    pl.semaphore_wait(xf_cr, 1)
                    pl.semaphore_wait(yf_cr, 1)
                xf[k] = pltpu.make_async_remote_copy(
                    yrecv.at[s3, pl.ds(0, H), :], dxrecv.at[s],
                    xf_ssem.at[s], xf_rsem.at[s],
                    device_id=xn, device_id_type=MESH)
                yf[k] = pltpu.make_async_remote_copy(
                    xrecv.at[s3, pl.ds(H, H), :], dyrecv.at[s],
                    yf_ssem.at[s], yf_rsem.at[s],
                    device_id=yn, device_id_type=MESH)
                xf[k].start()
                yf[k].start()
                st_r[k].wait()

            if 0 <= it - 3 < K:
                k, s = it - 3, (it - 3) % 2
                xf[k].wait_recv()
                yf[k].wait_recv()
                st_dx[k] = pltpu.make_async_copy(
                    dxrecv.at[s], out_hbm.at[pl.ds(q_dg * Q + k * C, H), :],
                    out_sems.at[3, s])
                st_dy[k] = pltpu.make_async_copy(
                    dyrecv.at[s],
                    out_hbm.at[pl.ds(q_dg * Q + k * C + H, H), :],
                    out_sems.at[4, s])
                st_dx[k].start()
                st_dy[k].start()
                xf[k].wait_send()
                yf[k].wait_send()
                st_x[k].wait()
                st_y[k].wait()
                st_dx[k].wait()
                st_dy[k].wait()
                if k + 3 < K:
                    sig(xr_cr, xn)
                    sig(yr_cr, yn)
                if k + 2 < K:
                    sig(xf_cr, xn)
                    sig(yf_cr, yn)

    return pl.pallas_call(
        body,
        out_shape=jax.ShapeDtypeStruct((m_per, n), jnp.bfloat16),
        in_specs=[pl.BlockSpec(memory_space=pl.ANY)],
        out_specs=pl.BlockSpec(memory_space=pl.ANY),
        scratch_shapes=[
            pltpu.VMEM((2, C, n), jnp.float32),
            pltpu.VMEM((2, C, n), jnp.bfloat16),
            pltpu.VMEM((2, C, n), jnp.bfloat16),
            pltpu.VMEM((2, C, n), jnp.bfloat16),
            pltpu.VMEM((3, C, n), jnp.bfloat16),
            pltpu.VMEM((3, C, n), jnp.bfloat16),
            pltpu.VMEM((2, C // 2, n), jnp.bfloat16),
            pltpu.VMEM((2, C // 2, n), jnp.bfloat16),
            pltpu.SemaphoreType.DMA((2,)),
            pltpu.SemaphoreType.DMA((5, 2)),
            pltpu.SemaphoreType.DMA((2,)),
            pltpu.SemaphoreType.DMA((2,)),
            pltpu.SemaphoreType.DMA((3,)),
            pltpu.SemaphoreType.DMA((3,)),
            pltpu.SemaphoreType.DMA((3,)),
            pltpu.SemaphoreType.DMA((3,)),
            pltpu.SemaphoreType.DMA((2,)),
            pltpu.SemaphoreType.DMA((2,)),
            pltpu.SemaphoreType.DMA((2,)),
            pltpu.SemaphoreType.DMA((2,)),
            pltpu.SemaphoreType.REGULAR,
            pltpu.SemaphoreType.REGULAR,
            pltpu.SemaphoreType.REGULAR,
            pltpu.SemaphoreType.REGULAR,
            pltpu.SemaphoreType.REGULAR,
        ],
        compiler_params=pltpu.CompilerParams(
            collective_id=0, vmem_limit_bytes=100 * 1024 * 1024),
    )(x)
